# baseline (speedup 1.0000x reference)
import jax
import jax.numpy as jnp
from jax import lax
from jax.experimental import pallas as pl
from jax.experimental.pallas import tpu as pltpu
from jax.experimental.pallas import tpu_sc as plsc

D = 1024
BATCH = 4
SEQ = 2048
NW = 32
S_PER_W = SEQ // NW
CHUNK = 8
N_G = S_PER_W // CHUNK
NRING = 3
LANES = 16


def _emb_body(ids_hbm, tok_hbm, pos_hbm, out_hbm, idx_v, *rest):
    tok_bufs = list(rest[0:12])
    pos_bufs = list(rest[12:15])
    gsems = list(rest[15:27])
    ssems = list(rest[27:39])
    psems = list(rest[39:42])

    c = lax.axis_index("c")
    s = lax.axis_index("s")
    wid = s * 2 + c
    seq0 = wid * S_PER_W

    id_copies = [
        pltpu.async_copy(
            ids_hbm.at[b, pl.ds(seq0, S_PER_W)],
            idx_v.at[pl.ds(b * S_PER_W, S_PER_W)], gsems[b])
        for b in range(BATCH)
    ]
    for cp in id_copies:
        cp.wait()

    def fire_pos(g):
        return pltpu.async_copy(
            pos_hbm.at[pl.ds(seq0 + g * CHUNK, CHUNK)],
            pos_bufs[g % 3], psems[g % 3])

    def fire_gather(g, b):
        i = (g % NRING) * BATCH + b
        return pltpu.async_copy(
            tok_hbm.at[idx_v.at[pl.ds(b * S_PER_W + g * CHUNK, CHUNK)]],
            tok_bufs[i], gsems[i])

    def fire_store(g, b):
        i = (g % NRING) * BATCH + b
        row = b * SEQ + seq0 + g * CHUNK
        return pltpu.async_copy(
            tok_bufs[i], out_hbm.at[pl.ds(row, CHUNK)], ssems[i])

    pend_p = {0: fire_pos(0), 1: fire_pos(1)}
    pend_g = {(g, b): fire_gather(g, b) for g in (0, 1) for b in range(BATCH)}
    pend_s = {}

    for g in range(N_G):
        if g + 2 < N_G:
            for b in range(BATCH):
                if g - 1 >= 0:
                    pend_s.pop((g - 1, b)).wait()
                pend_g[(g + 2, b)] = fire_gather(g + 2, b)
            pend_p[g + 2] = fire_pos(g + 2)
        pend_p.pop(g).wait()
        for b in range(BATCH):
            pend_g.pop((g, b)).wait()

        slot = g % NRING
        bufs = tok_bufs[slot * BATCH:(slot + 1) * BATCH]
        posb = pos_bufs[g % 3]

        def add_row(r, carry, _bufs=bufs, _pos=posb):
            for k in range(D // LANES):
                x = _pos[r, pl.ds(k * LANES, LANES)]
                for _t in _bufs:
                    plsc.addupdate(_t.at[r, pl.ds(k * LANES, LANES)], x)
            return carry

        lax.fori_loop(0, CHUNK, add_row, 0)

        for b in range(BATCH):
            pend_s[(g, b)] = fire_store(g, b)

    for key in sorted(pend_s):
        pend_s.pop(key).wait()


def kernel(input_ids, token_table, pos_table):
    ids_w = input_ids.astype(jnp.int32)
    mesh = plsc.VectorSubcoreMesh(core_axis_name="c", subcore_axis_name="s")
    run = pl.kernel(
        _emb_body,
        mesh=mesh,
        out_type=jax.ShapeDtypeStruct((BATCH * SEQ, D), jnp.float32),
        scratch_types=(
            [pltpu.VMEM((BATCH * S_PER_W,), jnp.int32)]
            + [pltpu.VMEM((CHUNK, D), jnp.float32)] * (NRING * BATCH)
            + [pltpu.VMEM((CHUNK, D), jnp.float32)] * 3
            + [pltpu.SemaphoreType.DMA] * (NRING * BATCH * 2 + 3)
        ),
    )
    out = run(ids_w, token_table, pos_table)
    return out.reshape(BATCH, SEQ, D)


# speedup vs baseline: 1.0400x; 1.0400x over previous
import jax
import jax.numpy as jnp
from jax import lax
from jax.experimental import pallas as pl
from jax.experimental.pallas import tpu as pltpu
from jax.experimental.pallas import tpu_sc as plsc

D = 1024
BATCH = 4
SEQ = 2048
NW = 32
S_PER_W = SEQ // NW
CHUNK = 16
N_CH = S_PER_W // CHUNK
N_JOBS = N_CH * BATCH
LANES = 16


def _emb_body(ids_hbm, tok_hbm, pos_hbm, out_hbm, idx_v, *rest):
    pos_bufs = list(rest[0:2])
    tok_bufs = list(rest[2:7])
    psems = list(rest[7:9])
    gsems = list(rest[9:14])
    ssems = list(rest[14:19])

    c = lax.axis_index("c")
    s = lax.axis_index("s")
    wid = s * 2 + c
    seq0 = wid * S_PER_W

    id_copies = [
        pltpu.async_copy(
            ids_hbm.at[b, pl.ds(seq0, S_PER_W)],
            idx_v.at[pl.ds(b * S_PER_W, S_PER_W)], gsems[b])
        for b in range(BATCH)
    ]
    for cp in id_copies:
        cp.wait()

    def fire_pos(ch):
        return pltpu.async_copy(
            pos_hbm.at[pl.ds(seq0 + ch * CHUNK, CHUNK)],
            pos_bufs[ch % 2], psems[ch % 2])

    def fire_gather(j):
        ch, b = divmod(j, BATCH)
        return pltpu.async_copy(
            tok_hbm.at[idx_v.at[pl.ds(b * S_PER_W + ch * CHUNK, CHUNK)]],
            tok_bufs[j % 5], gsems[j % 5])

    def fire_store(j):
        ch, b = divmod(j, BATCH)
        row = b * SEQ + seq0 + ch * CHUNK
        return pltpu.async_copy(
            tok_bufs[j % 5], out_hbm.at[pl.ds(row, CHUNK)], ssems[j % 5])

    pend_p = {0: fire_pos(0), 1: fire_pos(1)}
    pend_g = {j: fire_gather(j) for j in range(3)}
    pend_s = {}

    for j in range(N_JOBS):
        ch, b = divmod(j, BATCH)
        if j + 3 < N_JOBS:
            if j - 2 >= 0:
                pend_s.pop(j - 2).wait()
            pend_g[j + 3] = fire_gather(j + 3)
        if b == 0:
            pend_p.pop(ch).wait()
        pend_g.pop(j).wait()

        tok = tok_bufs[j % 5]
        posb = pos_bufs[ch % 2]

        def add_row(r, carry, _tok=tok, _pos=posb):
            for k in range(D // LANES):
                x = _pos[r, pl.ds(k * LANES, LANES)]
                plsc.addupdate(_tok.at[r, pl.ds(k * LANES, LANES)], x)
            return carry

        lax.fori_loop(0, CHUNK, add_row, 0)

        if b == BATCH - 1 and ch + 2 < N_CH:
            pend_p[ch + 2] = fire_pos(ch + 2)

        pend_s[j] = fire_store(j)

    for j in sorted(pend_s):
        pend_s[j].wait()


def kernel(input_ids, token_table, pos_table):
    ids_w = input_ids.astype(jnp.int32)
    mesh = plsc.VectorSubcoreMesh(core_axis_name="c", subcore_axis_name="s")
    run = pl.kernel(
        _emb_body,
        mesh=mesh,
        out_type=jax.ShapeDtypeStruct((BATCH * SEQ, D), jnp.float32),
        scratch_types=(
            [pltpu.VMEM((N_JOBS * CHUNK,), jnp.int32)]
            + [pltpu.VMEM((CHUNK, D), jnp.float32)] * 2
            + [pltpu.VMEM((CHUNK, D), jnp.float32)] * 5
            + [pltpu.SemaphoreType.DMA] * 12
        ),
    )
    out = run(ids_w, token_table, pos_table)
    return out.reshape(BATCH, SEQ, D)


# speedup vs baseline: 1.0435x; 1.0034x over previous
import jax
import jax.numpy as jnp
from jax import lax
from jax.experimental import pallas as pl
from jax.experimental.pallas import tpu as pltpu
from jax.experimental.pallas import tpu_sc as plsc

D = 1024
BATCH = 4
SEQ = 2048
NW = 32
S_PER_W = SEQ // NW
CHUNK = 16
N_CH = S_PER_W // CHUNK
N_JOBS = N_CH * BATCH
LANES = 16


def _emb_body(ids_hbm, tok_hbm, pos_hbm, out_hbm, idx_v,
              pos_ring, tok_ring, psem, gsem, ssem):
    pos_bufs = [pos_ring.at[pl.ds(i * CHUNK, CHUNK)] for i in range(2)]
    tok_bufs = [tok_ring.at[pl.ds(i * CHUNK, CHUNK)] for i in range(5)]
    psems = [psem.at[i] for i in range(2)]
    gsems = [gsem.at[i] for i in range(5)]
    ssems = [ssem.at[i] for i in range(5)]

    c = lax.axis_index("c")
    s = lax.axis_index("s")
    wid = s * 2 + c
    seq0 = wid * S_PER_W

    id_copies = [
        pltpu.async_copy(
            ids_hbm.at[b, pl.ds(seq0, S_PER_W)],
            idx_v.at[pl.ds(b * S_PER_W, S_PER_W)], gsems[b])
        for b in range(BATCH)
    ]
    for cp in id_copies:
        cp.wait()

    def fire_pos(ch):
        return pltpu.async_copy(
            pos_hbm.at[pl.ds(seq0 + ch * CHUNK, CHUNK)],
            pos_bufs[ch % 2], psems[ch % 2])

    def fire_gather(j):
        ch, b = divmod(j, BATCH)
        return pltpu.async_copy(
            tok_hbm.at[idx_v.at[pl.ds(b * S_PER_W + ch * CHUNK, CHUNK)]],
            tok_bufs[j % 5], gsems[j % 5])

    def fire_store(j):
        ch, b = divmod(j, BATCH)
        row = b * SEQ + seq0 + ch * CHUNK
        return pltpu.async_copy(
            tok_bufs[j % 5], out_hbm.at[pl.ds(row, CHUNK)], ssems[j % 5])

    pend_p = {0: fire_pos(0), 1: fire_pos(1)}
    pend_g = {j: fire_gather(j) for j in range(3)}
    pend_s = {}

    for j in range(N_JOBS):
        ch, b = divmod(j, BATCH)
        if j + 3 < N_JOBS:
            if j - 2 >= 0:
                pend_s.pop(j - 2).wait()
            pend_g[j + 3] = fire_gather(j + 3)
        if b == 0:
            pend_p.pop(ch).wait()
        pend_g.pop(j).wait()

        tok = tok_bufs[j % 5]
        posb = pos_bufs[ch % 2]

        def add_row(r, carry, _tok=tok, _pos=posb):
            for k in range(D // LANES):
                x = _pos[r, pl.ds(k * LANES, LANES)]
                plsc.addupdate(_tok.at[r, pl.ds(k * LANES, LANES)], x)
            return carry

        lax.fori_loop(0, CHUNK, add_row, 0)

        if b == BATCH - 1 and ch + 2 < N_CH:
            pend_p[ch + 2] = fire_pos(ch + 2)

        pend_s[j] = fire_store(j)

    for j in sorted(pend_s):
        pend_s[j].wait()


def kernel(input_ids, token_table, pos_table):
    ids_w = input_ids.astype(jnp.int32)
    mesh = plsc.VectorSubcoreMesh(core_axis_name="c", subcore_axis_name="s")
    run = pl.kernel(
        _emb_body,
        mesh=mesh,
        out_type=jax.ShapeDtypeStruct((BATCH * SEQ, D), jnp.float32),
        scratch_types=(
            [pltpu.VMEM((N_JOBS * CHUNK,), jnp.int32)]
            + [pltpu.VMEM((2 * CHUNK, D), jnp.float32)]
            + [pltpu.VMEM((5 * CHUNK, D), jnp.float32)]
            + [pltpu.SemaphoreType.DMA((2,)), pltpu.SemaphoreType.DMA((5,)),
               pltpu.SemaphoreType.DMA((5,))]
        ),
    )
    out = run(ids_w, token_table, pos_table)
    return out.reshape(BATCH, SEQ, D)
